# baseline (speedup 1.0000x reference)
import functools

import numpy as np
import jax
import jax.numpy as jnp
from jax import lax
from jax.experimental import pallas as pl
from jax.experimental.pallas import tpu as pltpu

GRID_RES = 0.1
D_MAX = 0.8
MASS = 10.0
GRAV = 9.81
VEL_MAX = 2.0
OMEGA_MAX = 2.0
DT = 0.01
N_TS = 8
STIFFNESS = 500.0
DAMPING = 50.0
FRICTION = 0.8
N_PTS_PER_TRACK = 8
P = 2 * N_PTS_PER_TRACK
LX, LY = 0.5, 0.3

_I_INV_NP = np.linalg.inv(np.diag(np.asarray((0.2, 0.3, 0.4), dtype=np.float64)))
I_INV = tuple(tuple(float(v) for v in row) for row in _I_INV_NP)

BBL = 4096


def _points_cols():
    xs = np.linspace(-LX / 2.0, LX / 2.0, N_PTS_PER_TRACK, dtype=np.float32)
    px = np.concatenate([xs, xs])
    py = np.concatenate([np.full_like(xs, -LY / 2.0), np.full_like(xs, +LY / 2.0)])
    sgn = np.concatenate([np.full_like(xs, -1.0), np.full_like(xs, +1.0)])
    return np.stack([px, py, sgn], axis=1)


def _rollout_kernel(H, W, n_ts, bbl,
                    z_ref, ctrl_ref, pts_ref, states_ref, forces_ref):
    HW = H * W
    HALF = HW // 2
    f32 = jnp.float32
    inv_res = 1.0 / GRID_RES
    mg = MASS * GRAV
    inv_m = 1.0 / MASS
    inv_P = 1.0 / float(P)
    half_Ly = 0.5 * LY
    dt = DT

    z_lo = z_ref[:, 0:HALF]
    z_hi = z_ref[:, HALF:HW]

    ptx = pts_ref[:, 0:1]
    pty = pts_ref[:, 1:2]
    sgn = pts_ref[:, 2:3]

    def interp(qx, qy):
        xi_f = (qx + D_MAX) * inv_res
        yi_f = (qy + D_MAX) * inv_res
        x_i = xi_f.astype(jnp.int32)
        y_i = yi_f.astype(jnp.int32)
        x_fr = xi_f - x_i.astype(f32)
        y_fr = yi_f - y_i.astype(f32)

        i_c = y_i + H * x_i
        idx4 = jnp.concatenate([i_c, i_c + H, i_c + 1, i_c + H + 1], axis=0)
        idx4 = jnp.clip(idx4, 0, HW - 1)

        idx_t = jnp.transpose(idx4, (1, 0))
        in_hi = idx_t >= HALF
        idx_m = jnp.bitwise_and(idx_t, HALF - 1)
        g_lo = jnp.take_along_axis(z_lo, idx_m, axis=1)
        g_hi = jnp.take_along_axis(z_hi, idx_m, axis=1)
        g = jnp.transpose(jnp.where(in_hi, g_hi, g_lo), (1, 0))

        zc = g[0:P]
        zf = g[P:2 * P]
        zl = g[2 * P:3 * P]
        zfl = g[3 * P:4 * P]

        w_c = (1.0 - x_fr) * (1.0 - y_fr)
        w_f = (1.0 - x_fr) * y_fr
        w_l = x_fr * (1.0 - y_fr)
        w_fl = x_fr * y_fr

        z_q = w_c * zc + w_f * zf + w_l * zl + w_fl * zfl
        dzdx = (zf - zc) * inv_res
        dzdy = (zl - zc) * inv_res
        wsum = w_c + w_f + w_l + w_fl
        return z_q, dzdx, dzdy, wsum

    zero = jnp.zeros((1, bbl), f32)
    one = zero + 1.0
    v0 = ctrl_ref[0:1, :]
    w0 = ctrl_ref[n_ts:n_ts + 1, :]

    zpb = jnp.zeros((P, bbl), f32)
    z_q0, dzdx0, dzdy0, wsum0 = interp(ptx + zpb, pty + zpb)
    x_x, x_y = zero, zero
    x_z = jnp.mean(z_q0, axis=0, keepdims=True)
    xd_x, xd_y, xd_z = v0, zero, zero
    om_x, om_y, om_z = zero, zero, w0
    r00, r01, r02 = one, zero, zero
    r10, r11, r12 = zero, one, zero
    r20, r21, r22 = zero, zero, one

    for t in range(n_ts):
        rx = r00 * ptx + r01 * pty
        ry = r10 * ptx + r11 * pty
        rz = r20 * ptx + r21 * pty
        pz_w = rz + x_z

        vx = xd_x + (om_y * rz - om_z * ry)
        vy = xd_y + (om_z * rx - om_x * rz)
        vz = xd_z + (om_x * ry - om_y * rx)

        if t == 0:
            z_q, dzdx, dzdy, wsum = z_q0, dzdx0, dzdy0, wsum0
        else:
            qx = rx + x_x
            qy = ry + x_y
            z_q, dzdx, dzdy, wsum = interp(qx, qy)
        stiff_q = STIFFNESS * wsum
        damp_q = DAMPING * wsum
        fric_q = FRICTION * wsum

        inv_n = lax.rsqrt(dzdx * dzdx + dzdy * dzdy + 1.0)
        nx = -dzdx * inv_n
        ny = -dzdy * inv_n
        nz = inv_n

        dh = pz_w - z_q
        in_contact = jax.nn.sigmoid(-10.0 * dh)
        v_n = vx * nx + vy * ny + vz * nz

        f_scale = -(stiff_q * dh + damp_q * v_n) * in_contact * inv_P
        fs_x = jnp.clip(f_scale * nx, -mg, mg)
        fs_y = jnp.clip(f_scale * ny, -mg, mg)
        fs_z = jnp.clip(f_scale * nz, -mg, mg)
        n_mag = jnp.sqrt(fs_x * fs_x + fs_y * fs_y + fs_z * fs_z)

        t_inv = lax.rsqrt(jnp.maximum(r00 * r00 + r10 * r10 + r20 * r20, 1e-12))
        td_x = r00 * t_inv
        td_y = r10 * t_inv
        td_z = r20 * t_inv

        v_cmd = ctrl_ref[t:t + 1, :]
        w_cmd = ctrl_ref[n_ts + t:n_ts + t + 1, :]
        coef = v_cmd + sgn * (half_Ly * w_cmd)
        mu_x = jnp.clip(fric_q * (coef * td_x), -VEL_MAX, VEL_MAX)
        mu_y = jnp.clip(fric_q * (coef * td_y), -VEL_MAX, VEL_MAX)
        mu_z = jnp.clip(fric_q * (coef * td_z), -VEL_MAX, VEL_MAX)

        dvx = mu_x - vx
        dvy = mu_y - vy
        dvz = mu_z - vz
        dv_n = dvx * nx + dvy * ny + dvz * nz
        ff_x = jnp.clip(n_mag * (dvx - dv_n * nx), -mg, mg)
        ff_y = jnp.clip(n_mag * (dvy - dv_n * ny), -mg, mg)
        ff_z = jnp.clip(n_mag * (dvz - dv_n * nz), -mg, mg)

        fx = fs_x + ff_x
        fy = fs_y + ff_y
        fz = fs_z + ff_z
        tq_x = jnp.sum(ry * fz - rz * fy, axis=0, keepdims=True)
        tq_y = jnp.sum(rz * fx - rx * fz, axis=0, keepdims=True)
        tq_z = jnp.sum(rx * fy - ry * fx, axis=0, keepdims=True)

        od_x = jnp.clip(I_INV[0][0] * tq_x + I_INV[0][1] * tq_y + I_INV[0][2] * tq_z,
                        -OMEGA_MAX, OMEGA_MAX)
        od_y = jnp.clip(I_INV[1][0] * tq_x + I_INV[1][1] * tq_y + I_INV[1][2] * tq_z,
                        -OMEGA_MAX, OMEGA_MAX)
        od_z = jnp.clip(I_INV[2][0] * tq_x + I_INV[2][1] * tq_y + I_INV[2][2] * tq_z,
                        -OMEGA_MAX, OMEGA_MAX)

        xdd_x = (0.0 + jnp.sum(fx, axis=0, keepdims=True)) * inv_m
        xdd_y = (0.0 + jnp.sum(fy, axis=0, keepdims=True)) * inv_m
        xdd_z = (-mg + jnp.sum(fz, axis=0, keepdims=True)) * inv_m

        xd_x = xd_x + xdd_x * dt
        xd_y = xd_y + xdd_y * dt
        xd_z = xd_z + xdd_z * dt
        x_x = x_x + xd_x * dt
        x_y = x_y + xd_y * dt
        x_z = x_z + xd_z * dt
        om_x = om_x + od_x * dt
        om_y = om_y + od_y * dt
        om_z = om_z + od_z * dt

        theta = jnp.sqrt(om_x * om_x + om_y * om_y + om_z * om_z)
        inv_th = 1.0 / jnp.maximum(theta, 1e-6)
        ux = om_x * inv_th
        uy = om_y * inv_th
        uz = om_z * inv_th
        uu = ux * ux + uy * uy + uz * uz
        s = jnp.sin(theta * dt)
        c1 = 1.0 - jnp.cos(theta * dt)

        a00 = 1.0 + c1 * (ux * ux - uu)
        a01 = -s * uz + c1 * (ux * uy)
        a02 = s * uy + c1 * (ux * uz)
        a10 = s * uz + c1 * (uy * ux)
        a11 = 1.0 + c1 * (uy * uy - uu)
        a12 = -s * ux + c1 * (uy * uz)
        a20 = -s * uy + c1 * (uz * ux)
        a21 = s * ux + c1 * (uz * uy)
        a22 = 1.0 + c1 * (uz * uz - uu)

        n00 = r00 * a00 + r01 * a10 + r02 * a20
        n01 = r00 * a01 + r01 * a11 + r02 * a21
        n02 = r00 * a02 + r01 * a12 + r02 * a22
        n10 = r10 * a00 + r11 * a10 + r12 * a20
        n11 = r10 * a01 + r11 * a11 + r12 * a21
        n12 = r10 * a02 + r11 * a12 + r12 * a22
        n20 = r20 * a00 + r21 * a10 + r22 * a20
        n21 = r20 * a01 + r21 * a11 + r22 * a21
        n22 = r20 * a02 + r21 * a12 + r22 * a22
        r00, r01, r02 = n00, n01, n02
        r10, r11, r12 = n10, n11, n12
        r20, r21, r22 = n20, n21, n22

        state_t = jnp.concatenate(
            [x_x, x_y, x_z, xd_x, xd_y, xd_z, om_x, om_y, om_z,
             r00, r01, r02, r10, r11, r12, r20, r21, r22], axis=0)
        force_t = jnp.concatenate(
            [fs_x, fs_y, fs_z, ff_x, ff_y, ff_z], axis=0)
        states_ref[pl.ds(18 * t, 18), :] = state_t
        forces_ref[pl.ds(6 * P * t, 6 * P), :] = force_t


def kernel(z_grid, controls):
    B, H, W = z_grid.shape
    n_ts = min(N_TS, controls.shape[1])

    z_flat = z_grid.astype(jnp.float32).reshape(B, H * W)
    ctrl = controls[:, :n_ts].astype(jnp.float32)
    ctrl_t = jnp.concatenate([ctrl[:, :, 0], ctrl[:, :, 1]], axis=1).T

    bbl = BBL
    nb = -(-B // bbl)
    Bp = nb * bbl
    if Bp != B:
        z_flat = jnp.pad(z_flat, ((0, Bp - B), (0, 0)))
        ctrl_t = jnp.pad(ctrl_t, ((0, 0), (0, Bp - B)))

    pts = jnp.asarray(_points_cols())

    kfn = functools.partial(_rollout_kernel, H, W, n_ts, bbl)
    statesT, forcesT = pl.pallas_call(
        kfn,
        grid=(nb,),
        in_specs=[
            pl.BlockSpec((bbl, H * W), lambda i: (i, 0)),
            pl.BlockSpec((2 * n_ts, bbl), lambda i: (0, i)),
            pl.BlockSpec((P, 3), lambda i: (0, 0)),
        ],
        out_specs=[
            pl.BlockSpec((18 * n_ts, bbl), lambda i: (0, i)),
            pl.BlockSpec((6 * P * n_ts, bbl), lambda i: (0, i)),
        ],
        out_shape=(
            jax.ShapeDtypeStruct((18 * n_ts, Bp), jnp.float32),
            jax.ShapeDtypeStruct((6 * P * n_ts, Bp), jnp.float32),
        ),
        compiler_params=pltpu.CompilerParams(
            dimension_semantics=("parallel",)),
    )(z_flat, ctrl_t, pts)

    S4 = statesT.reshape(n_ts, 18, Bp)
    S = jnp.transpose(S4, (2, 0, 1))[:B]
    F4 = forcesT.reshape(n_ts, 6, P, Bp)
    F_springs = jnp.transpose(F4[:, 0:3], (3, 0, 2, 1))[:B]
    F_frictions = jnp.transpose(F4[:, 3:6], (3, 0, 2, 1))[:B]

    Xs = S[:, :, 0:3]
    Xds = S[:, :, 3:6]
    Omegas = S[:, :, 6:9]
    Rs = S[:, :, 9:18].reshape(B, n_ts, 3, 3)

    delta_h = jnp.float32(MASS * GRAV) / (jnp.float32(STIFFNESS) + jnp.float32(1e-6))
    Xs = Xs + Rs[:, :, :, 2] * delta_h

    return (Xs, Xds, Rs, Omegas), (F_springs, F_frictions)


# speedup vs baseline: 1.1824x; 1.1824x over previous
import functools

import numpy as np
import jax
import jax.numpy as jnp
from jax import lax
from jax.experimental import pallas as pl
from jax.experimental.pallas import tpu as pltpu

GRID_RES = 0.1
D_MAX = 0.8
MASS = 10.0
GRAV = 9.81
VEL_MAX = 2.0
OMEGA_MAX = 2.0
DT = 0.01
N_TS = 8
STIFFNESS = 500.0
DAMPING = 50.0
FRICTION = 0.8
N_PTS_PER_TRACK = 8
P = 2 * N_PTS_PER_TRACK
LX, LY = 0.5, 0.3

_I_INV_NP = np.linalg.inv(np.diag(np.asarray((0.2, 0.3, 0.4), dtype=np.float64)))
I_INV = tuple(tuple(float(v) for v in row) for row in _I_INV_NP)

BBL = 2048


def _points_cols():
    xs = np.linspace(-LX / 2.0, LX / 2.0, N_PTS_PER_TRACK, dtype=np.float32)
    px = np.concatenate([xs, xs])
    py = np.concatenate([np.full_like(xs, -LY / 2.0), np.full_like(xs, +LY / 2.0)])
    sgn = np.concatenate([np.full_like(xs, -1.0), np.full_like(xs, +1.0)])
    return np.stack([px, py, sgn], axis=1)


def _rollout_kernel(H, W, n_ts, bbl,
                    z_ref, ctrl_ref, pts_ref, states_ref, forces_ref):
    HW = H * W
    HALF = HW // 2
    f32 = jnp.float32
    inv_res = 1.0 / GRID_RES
    mg = MASS * GRAV
    inv_m = 1.0 / MASS
    inv_P = 1.0 / float(P)
    half_Ly = 0.5 * LY
    dt = DT

    z_lo = z_ref[:, 0:HALF]
    z_hi = z_ref[:, HALF:HW]

    ptx = pts_ref[:, 0:1]
    pty = pts_ref[:, 1:2]
    sgn = pts_ref[:, 2:3]

    def interp(qx, qy):
        xi_f = (qx + D_MAX) * inv_res
        yi_f = (qy + D_MAX) * inv_res
        x_i = xi_f.astype(jnp.int32)
        y_i = yi_f.astype(jnp.int32)
        x_fr = xi_f - x_i.astype(f32)
        y_fr = yi_f - y_i.astype(f32)

        i_c = y_i + H * x_i
        idx4 = jnp.concatenate([i_c, i_c + H, i_c + 1, i_c + H + 1], axis=0)
        idx4 = jnp.clip(idx4, 0, HW - 1)

        idx_t = jnp.transpose(idx4, (1, 0))
        in_hi = idx_t >= HALF
        idx_m = jnp.bitwise_and(idx_t, HALF - 1)
        g_lo = jnp.take_along_axis(z_lo, idx_m, axis=1)
        g_hi = jnp.take_along_axis(z_hi, idx_m, axis=1)
        g = jnp.transpose(jnp.where(in_hi, g_hi, g_lo), (1, 0))

        zc = g[0:P]
        zf = g[P:2 * P]
        zl = g[2 * P:3 * P]
        zfl = g[3 * P:4 * P]

        w_c = (1.0 - x_fr) * (1.0 - y_fr)
        w_f = (1.0 - x_fr) * y_fr
        w_l = x_fr * (1.0 - y_fr)
        w_fl = x_fr * y_fr

        z_q = w_c * zc + w_f * zf + w_l * zl + w_fl * zfl
        dzdx = (zf - zc) * inv_res
        dzdy = (zl - zc) * inv_res
        return z_q, dzdx, dzdy

    zero = jnp.zeros((1, bbl), f32)
    one = zero + 1.0
    v0 = ctrl_ref[0:1, :]
    w0 = ctrl_ref[n_ts:n_ts + 1, :]

    zpb = jnp.zeros((P, bbl), f32)
    z_q0, dzdx0, dzdy0 = interp(ptx + zpb, pty + zpb)
    x_x, x_y = zero, zero
    x_z = jnp.mean(z_q0, axis=0, keepdims=True)
    xd_x, xd_y, xd_z = v0, zero, zero
    om_x, om_y, om_z = zero, zero, w0
    r00, r01, r02 = one, zero, zero
    r10, r11, r12 = zero, one, zero
    r20, r21, r22 = zero, zero, one

    for t in range(n_ts):
        rx = r00 * ptx + r01 * pty
        ry = r10 * ptx + r11 * pty
        rz = r20 * ptx + r21 * pty
        pz_w = rz + x_z

        vx = xd_x + (om_y * rz - om_z * ry)
        vy = xd_y + (om_z * rx - om_x * rz)
        vz = xd_z + (om_x * ry - om_y * rx)

        if t == 0:
            z_q, dzdx, dzdy = z_q0, dzdx0, dzdy0
        else:
            qx = rx + x_x
            qy = ry + x_y
            z_q, dzdx, dzdy = interp(qx, qy)

        inv_n = lax.rsqrt(dzdx * dzdx + dzdy * dzdy + 1.0)
        nx = -dzdx * inv_n
        ny = -dzdy * inv_n
        nz = inv_n

        dh = pz_w - z_q
        in_contact = jax.nn.sigmoid(-10.0 * dh)
        v_n = vx * nx + vy * ny + vz * nz

        f_scale = (dh * (-STIFFNESS * inv_P) + v_n * (-DAMPING * inv_P)) * in_contact
        fs_x = jnp.clip(f_scale * nx, -mg, mg)
        fs_y = jnp.clip(f_scale * ny, -mg, mg)
        fs_z = jnp.clip(f_scale * nz, -mg, mg)
        n_mag = jnp.sqrt(fs_x * fs_x + fs_y * fs_y + fs_z * fs_z)

        t_inv = lax.rsqrt(jnp.maximum(r00 * r00 + r10 * r10 + r20 * r20, 1e-12))
        td_x = r00 * t_inv
        td_y = r10 * t_inv
        td_z = r20 * t_inv

        v_cmd = ctrl_ref[t:t + 1, :]
        w_cmd = ctrl_ref[n_ts + t:n_ts + t + 1, :]
        coef = FRICTION * (v_cmd + sgn * (half_Ly * w_cmd))
        mu_x = jnp.clip(coef * td_x, -VEL_MAX, VEL_MAX)
        mu_y = jnp.clip(coef * td_y, -VEL_MAX, VEL_MAX)
        mu_z = jnp.clip(coef * td_z, -VEL_MAX, VEL_MAX)

        dvx = mu_x - vx
        dvy = mu_y - vy
        dvz = mu_z - vz
        dv_n = dvx * nx + dvy * ny + dvz * nz
        ff_x = jnp.clip(n_mag * (dvx - dv_n * nx), -mg, mg)
        ff_y = jnp.clip(n_mag * (dvy - dv_n * ny), -mg, mg)
        ff_z = jnp.clip(n_mag * (dvz - dv_n * nz), -mg, mg)

        fx = fs_x + ff_x
        fy = fs_y + ff_y
        fz = fs_z + ff_z
        tq_x = jnp.sum(ry * fz - rz * fy, axis=0, keepdims=True)
        tq_y = jnp.sum(rz * fx - rx * fz, axis=0, keepdims=True)
        tq_z = jnp.sum(rx * fy - ry * fx, axis=0, keepdims=True)

        od_x = jnp.clip(I_INV[0][0] * tq_x + I_INV[0][1] * tq_y + I_INV[0][2] * tq_z,
                        -OMEGA_MAX, OMEGA_MAX)
        od_y = jnp.clip(I_INV[1][0] * tq_x + I_INV[1][1] * tq_y + I_INV[1][2] * tq_z,
                        -OMEGA_MAX, OMEGA_MAX)
        od_z = jnp.clip(I_INV[2][0] * tq_x + I_INV[2][1] * tq_y + I_INV[2][2] * tq_z,
                        -OMEGA_MAX, OMEGA_MAX)

        xdd_x = (0.0 + jnp.sum(fx, axis=0, keepdims=True)) * inv_m
        xdd_y = (0.0 + jnp.sum(fy, axis=0, keepdims=True)) * inv_m
        xdd_z = (-mg + jnp.sum(fz, axis=0, keepdims=True)) * inv_m

        xd_x = xd_x + xdd_x * dt
        xd_y = xd_y + xdd_y * dt
        xd_z = xd_z + xdd_z * dt
        x_x = x_x + xd_x * dt
        x_y = x_y + xd_y * dt
        x_z = x_z + xd_z * dt
        om_x = om_x + od_x * dt
        om_y = om_y + od_y * dt
        om_z = om_z + od_z * dt

        theta = jnp.sqrt(om_x * om_x + om_y * om_y + om_z * om_z)
        inv_th = 1.0 / jnp.maximum(theta, 1e-6)
        ux = om_x * inv_th
        uy = om_y * inv_th
        uz = om_z * inv_th
        uu = ux * ux + uy * uy + uz * uz
        s = jnp.sin(theta * dt)
        c1 = 1.0 - jnp.cos(theta * dt)

        a00 = 1.0 + c1 * (ux * ux - uu)
        a01 = -s * uz + c1 * (ux * uy)
        a02 = s * uy + c1 * (ux * uz)
        a10 = s * uz + c1 * (uy * ux)
        a11 = 1.0 + c1 * (uy * uy - uu)
        a12 = -s * ux + c1 * (uy * uz)
        a20 = -s * uy + c1 * (uz * ux)
        a21 = s * ux + c1 * (uz * uy)
        a22 = 1.0 + c1 * (uz * uz - uu)

        n00 = r00 * a00 + r01 * a10 + r02 * a20
        n01 = r00 * a01 + r01 * a11 + r02 * a21
        n02 = r00 * a02 + r01 * a12 + r02 * a22
        n10 = r10 * a00 + r11 * a10 + r12 * a20
        n11 = r10 * a01 + r11 * a11 + r12 * a21
        n12 = r10 * a02 + r11 * a12 + r12 * a22
        n20 = r20 * a00 + r21 * a10 + r22 * a20
        n21 = r20 * a01 + r21 * a11 + r22 * a21
        n22 = r20 * a02 + r21 * a12 + r22 * a22
        r00, r01, r02 = n00, n01, n02
        r10, r11, r12 = n10, n11, n12
        r20, r21, r22 = n20, n21, n22

        state_t = jnp.concatenate(
            [x_x, x_y, x_z, xd_x, xd_y, xd_z, om_x, om_y, om_z,
             r00, r01, r02, r10, r11, r12, r20, r21, r22], axis=0)
        force_t = jnp.concatenate(
            [fs_x, fs_y, fs_z, ff_x, ff_y, ff_z], axis=0)
        states_ref[pl.ds(18 * t, 18), :] = state_t
        forces_ref[pl.ds(6 * P * t, 6 * P), :] = force_t


def kernel(z_grid, controls):
    B, H, W = z_grid.shape
    n_ts = min(N_TS, controls.shape[1])

    z_flat = z_grid.astype(jnp.float32).reshape(B, H * W)
    ctrl = controls[:, :n_ts].astype(jnp.float32)
    ctrl_t = jnp.concatenate([ctrl[:, :, 0], ctrl[:, :, 1]], axis=1).T

    bbl = BBL
    nb = -(-B // bbl)
    Bp = nb * bbl
    if Bp != B:
        z_flat = jnp.pad(z_flat, ((0, Bp - B), (0, 0)))
        ctrl_t = jnp.pad(ctrl_t, ((0, 0), (0, Bp - B)))

    pts = jnp.asarray(_points_cols())

    kfn = functools.partial(_rollout_kernel, H, W, n_ts, bbl)
    statesT, forcesT = pl.pallas_call(
        kfn,
        grid=(nb,),
        in_specs=[
            pl.BlockSpec((bbl, H * W), lambda i: (i, 0)),
            pl.BlockSpec((2 * n_ts, bbl), lambda i: (0, i)),
            pl.BlockSpec((P, 3), lambda i: (0, 0)),
        ],
        out_specs=[
            pl.BlockSpec((18 * n_ts, bbl), lambda i: (0, i)),
            pl.BlockSpec((6 * P * n_ts, bbl), lambda i: (0, i)),
        ],
        out_shape=(
            jax.ShapeDtypeStruct((18 * n_ts, Bp), jnp.float32),
            jax.ShapeDtypeStruct((6 * P * n_ts, Bp), jnp.float32),
        ),
        compiler_params=pltpu.CompilerParams(
            dimension_semantics=("parallel",)),
    )(z_flat, ctrl_t, pts)

    S4 = statesT.reshape(n_ts, 18, Bp)
    S = jnp.transpose(S4, (2, 0, 1))[:B]
    F4 = forcesT.reshape(n_ts, 6, P, Bp)
    F_springs = jnp.transpose(F4[:, 0:3], (3, 0, 2, 1))[:B]
    F_frictions = jnp.transpose(F4[:, 3:6], (3, 0, 2, 1))[:B]

    Xs = S[:, :, 0:3]
    Xds = S[:, :, 3:6]
    Omegas = S[:, :, 6:9]
    Rs = S[:, :, 9:18].reshape(B, n_ts, 3, 3)

    delta_h = jnp.float32(MASS * GRAV) / (jnp.float32(STIFFNESS) + jnp.float32(1e-6))
    Xs = Xs + Rs[:, :, :, 2] * delta_h

    return (Xs, Xds, Rs, Omegas), (F_springs, F_frictions)
